# baseline (speedup 1.0000x reference)
import jax
import jax.numpy as jnp
from jax import lax
from jax.experimental import pallas as pl
from jax.experimental.pallas import tpu as pltpu
from jax.experimental.pallas import tpu_sc as plsc

_B = 16384
_XD = 64
_D = 10
_DP = 16
_NT = 5
_OW = _XD + _NT * _D

_NC = 2
_NS = 16
_NW = _NC * _NS
_R = 128
_NPASS = _B // (_NW * _R)
_NBLK = _B // _R
_OB = _R * _OW
_UNROLL = 4


def _body(x_hbm, yb_hbm, t0, t1, t2, t3, t4, out_hbm, xv, idxv, rows, outv, sem):
    wid = lax.axis_index("s") * _NC + lax.axis_index("c")
    tables = (t0, t1, t2, t3, t4)

    for p in range(_NPASS):
        blk = wid * _NPASS + p
        with jax.named_scope("stage_in"):
            pltpu.sync_copy(x_hbm.at[pl.ds(blk * (_R * _XD), _R * _XD)], xv)
            pltpu.sync_copy(yb_hbm.at[blk], idxv)
        with jax.named_scope("gather"):
            cps = [
                pltpu.make_async_copy(tables[i].at[idxv.at[i]], rows.at[i], sem)
                for i in range(_NT)
            ]
            for c in cps:
                c.start()
            for c in cps:
                c.wait()

        def assemble(it, _):
            for u in range(_UNROLL):
                r = it * _UNROLL + u
                ob = r * _OW
                xb = r * _XD
                for c in range(_XD // 16):
                    outv[pl.ds(ob + 16 * c, 16)] = xv[pl.ds(xb + 16 * c, 16)]
                for i in range(_NT):
                    outv[pl.ds(ob + _XD + _D * i, 16)] = rows[i, r, :]
            return ()

        with jax.named_scope("assemble"):
            lax.fori_loop(0, _R // _UNROLL, assemble, (), unroll=2)

        with jax.named_scope("writeout"):
            pltpu.sync_copy(outv.at[pl.ds(0, _OB)], out_hbm.at[pl.ds(blk * _OB, _OB)])


_sc_call = pl.kernel(
    _body,
    out_type=jax.ShapeDtypeStruct((_B * _OW,), jnp.float32),
    mesh=plsc.VectorSubcoreMesh(core_axis_name="c", subcore_axis_name="s"),
    scratch_types=[
        pltpu.VMEM((_R * _XD,), jnp.float32),
        pltpu.VMEM((_NT, _R), jnp.int32),
        pltpu.VMEM((_NT, _R, _DP), jnp.float32),
        pltpu.VMEM((_OB + 16,), jnp.float32),
        pltpu.SemaphoreType.DMA,
    ],
    compiler_params=pltpu.CompilerParams(use_tc_tiling_on_sc=False),
)


def kernel(x, y, emb0, emb1, emb2, emb3, emb4):
    yb = y.astype(jnp.int32).reshape(_NBLK, _R, _NT).transpose(0, 2, 1)
    tables = [
        jnp.pad(t, ((0, 0), (0, _DP - _D)))
        for t in (emb0, emb1, emb2, emb3, emb4)
    ]
    out = _sc_call(x.reshape(-1), yb, *tables)
    return out.reshape(_B, _OW)


# speedup vs baseline: 2.2696x; 2.2696x over previous
import jax
import jax.numpy as jnp
from jax import lax
from jax.experimental import pallas as pl
from jax.experimental.pallas import tpu as pltpu
from jax.experimental.pallas import tpu_sc as plsc

_VOCABS = (6, 7, 12, 7, 96)
_B = 16384
_XD = 64
_D = 10
_DP = 16
_NT = 5
_OW = _XD + _NT * _D
_CV = sum(_VOCABS)

_NC = 2
_NS = 16
_NW = _NC * _NS
_BPW = _B // _NW
_OB = _BPW * _OW
_UNROLL = 4

_TBASE = []
_acc = 0
for _v in _VOCABS:
    _TBASE.append(_acc * _DP)
    _acc += _v


def _body(x_hbm, yf_hbm, tcat_hbm, out_hbm, xv, yv, tv, outv, sem):
    wid = lax.axis_index("s") * _NC + lax.axis_index("c")

    pltpu.sync_copy(tcat_hbm, tv)
    pltpu.sync_copy(x_hbm.at[pl.ds(wid * (_BPW * _XD), _BPW * _XD)], xv)
    pltpu.sync_copy(
        yf_hbm.at[pl.ds(wid * (_BPW * _NT), _BPW * _NT)],
        yv.at[pl.ds(0, _BPW * _NT)],
    )

    lane = lax.iota(jnp.int32, 16)
    biases = [lane + _TBASE[i] for i in range(_NT)]

    def assemble(it, _):
        for u in range(_UNROLL):
            r = it * _UNROLL + u
            ob = r * _OW
            xb = r * _XD
            for c in range(_XD // 16):
                outv[pl.ds(ob + 16 * c, 16)] = xv[pl.ds(xb + 16 * c, 16)]
            yrow = yv[pl.ds(r * _NT, 16)]
            for i in range(_NT):
                yi = yrow[i]
                row = plsc.load_gather(tv, [yi * _DP + biases[i]])
                outv[pl.ds(ob + _XD + _D * i, 16)] = row
        return ()

    lax.fori_loop(0, _BPW // _UNROLL, assemble, (), unroll=2)

    pltpu.sync_copy(outv.at[pl.ds(0, _OB)], out_hbm.at[pl.ds(wid * _OB, _OB)])


_sc_call = pl.kernel(
    _body,
    out_type=jax.ShapeDtypeStruct((_B * _OW,), jnp.float32),
    mesh=plsc.VectorSubcoreMesh(core_axis_name="c", subcore_axis_name="s"),
    scratch_types=[
        pltpu.VMEM((_BPW * _XD,), jnp.float32),
        pltpu.VMEM((_BPW * _NT + 16,), jnp.int32),
        pltpu.VMEM((_CV * _DP,), jnp.float32),
        pltpu.VMEM((_OB + 16,), jnp.float32),
        pltpu.SemaphoreType.DMA,
    ],
    compiler_params=pltpu.CompilerParams(use_tc_tiling_on_sc=False, needs_layout_passes=False),
)


def kernel(x, y, emb0, emb1, emb2, emb3, emb4):
    yf = y.astype(jnp.int32).reshape(-1)
    tcat = jnp.concatenate([emb0, emb1, emb2, emb3, emb4], axis=0)
    tcat = jnp.pad(tcat, ((0, 0), (0, _DP - _D))).reshape(-1)
    out = _sc_call(x.reshape(-1), yf, tcat)
    return out.reshape(_B, _OW)


# speedup vs baseline: 2.9127x; 1.2833x over previous
import jax
import jax.numpy as jnp
from jax import lax
from jax.experimental import pallas as pl
from jax.experimental.pallas import tpu as pltpu
from jax.experimental.pallas import tpu_sc as plsc

_VOCABS = (6, 7, 12, 7, 96)
_B = 16384
_XD = 64
_D = 10
_DP = 16
_NT = 5
_OW = _XD + _NT * _D
_CV = sum(_VOCABS)

_NC = 2
_NS = 16
_NW = _NC * _NS
_BPW = _B // _NW
_SLAB = 256
_NSLAB = _BPW // _SLAB
_UNROLL = 4

_TBASE = []
_acc = 0
for _v in _VOCABS:
    _TBASE.append(_acc * _DP)
    _acc += _v


def _body(x_hbm, yf_hbm, tcat_hbm, out_hbm, xv, yv, tv, outv, sem):
    wid = lax.axis_index("s") * _NC + lax.axis_index("c")

    pltpu.sync_copy(tcat_hbm, tv)
    pltpu.sync_copy(
        yf_hbm.at[pl.ds(wid * (_BPW * _NT), _BPW * _NT)],
        yv.at[pl.ds(0, _BPW * _NT)],
    )

    lane = lax.iota(jnp.int32, 16)
    biases = [lane + _TBASE[i] for i in range(_NT)]
    tail_sel = lane < 6
    tail3 = lane + (_TBASE[3] + 4)
    tail4 = lane + (_TBASE[4] - 6)

    for s in range(_NSLAB):
        base = wid * _BPW + s * _SLAB
        pltpu.sync_copy(x_hbm.at[pl.ds(base, _SLAB), :], xv)

        def assemble(it, _):
            for u in range(_UNROLL):
                r = it * _UNROLL + u
                for c in range(_XD // 16):
                    outv[r, pl.ds(16 * c, 16)] = xv[r, pl.ds(16 * c, 16)]
                yrow = yv[pl.ds((s * _SLAB + r) * _NT, 16)]
                for i in range(_NT - 1):
                    yi = yrow[i]
                    row = plsc.load_gather(tv, [yi * _DP + biases[i]])
                    outv[r, pl.ds(_XD + _D * i, 16)] = row
                tidx = jnp.where(
                    tail_sel, yrow[3] * _DP + tail3, yrow[4] * _DP + tail4
                )
                outv[r, pl.ds(_XD + 34, 16)] = plsc.load_gather(tv, [tidx])
            return ()

        lax.fori_loop(0, _SLAB // _UNROLL, assemble, (), unroll=2)

        pltpu.sync_copy(outv, out_hbm.at[pl.ds(base, _SLAB), :])


_sc_call = pl.kernel(
    _body,
    out_type=jax.ShapeDtypeStruct((_B, _OW), jnp.float32),
    mesh=plsc.VectorSubcoreMesh(core_axis_name="c", subcore_axis_name="s"),
    scratch_types=[
        pltpu.VMEM((_SLAB, _XD), jnp.float32),
        pltpu.VMEM((_BPW * _NT + 16,), jnp.int32),
        pltpu.VMEM((_CV * _DP,), jnp.float32),
        pltpu.VMEM((_SLAB, _OW), jnp.float32),
        pltpu.SemaphoreType.DMA,
    ],
    compiler_params=pltpu.CompilerParams(
        use_tc_tiling_on_sc=True,
        needs_layout_passes=False,
        disable_bounds_checks=True,
    ),
)


def kernel(x, y, emb0, emb1, emb2, emb3, emb4):
    yf = y.astype(jnp.int32).reshape(-1)
    tcat = jnp.concatenate([emb0, emb1, emb2, emb3, emb4], axis=0)
    tcat = jnp.pad(tcat, ((0, 0), (0, _DP - _D))).reshape(-1)
    return _sc_call(x, yf, tcat)


# speedup vs baseline: 5.3543x; 1.8382x over previous
import jax
import jax.numpy as jnp
from jax import lax
from jax.experimental import pallas as pl
from jax.experimental.pallas import tpu as pltpu
from jax.experimental.pallas import tpu_sc as plsc

_VOCABS = (6, 7, 12, 7, 96)
_B = 16384
_XD = 64
_D = 10
_DP = 16
_NT = 5
_OW = _XD + _NT * _D
_CV = sum(_VOCABS)

_NC = 2
_NS = 16
_NW = _NC * _NS
_BPW = _B // _NW
_NG = _BPW // 16
_UNROLL = 2

_TBASE = []
_acc = 0
for _v in _VOCABS:
    _TBASE.append(_acc * _DP)
    _acc += _v


def _body(xt_hbm, yt_hbm, tcat_hbm, ot_hbm, yv, tv, otv, sem):
    wid = lax.axis_index("s") * _NC + lax.axis_index("c")
    b0 = wid * _BPW

    pltpu.sync_copy(tcat_hbm, tv)
    pltpu.sync_copy(yt_hbm.at[:, pl.ds(b0, _BPW)], yv)
    pltpu.sync_copy(xt_hbm.at[:, pl.ds(b0, _BPW)], otv.at[pl.ds(0, _XD), :])

    def assemble(it, _):
        for u in range(_UNROLL):
            p = it * _UNROLL + u
            for i in range(_NT):
                y16 = yv[i, pl.ds(16 * p, 16)]
                base = y16 * _DP + _TBASE[i]
                for j in range(_D):
                    col = plsc.load_gather(tv, [base + j])
                    otv[_XD + _D * i + j, pl.ds(16 * p, 16)] = col
        return ()

    lax.fori_loop(0, _NG // _UNROLL, assemble, (), unroll=2)

    pltpu.sync_copy(otv, ot_hbm.at[:, pl.ds(b0, _BPW)])


_sc_call = pl.kernel(
    _body,
    out_type=jax.ShapeDtypeStruct((_OW, _B), jnp.float32),
    mesh=plsc.VectorSubcoreMesh(core_axis_name="c", subcore_axis_name="s"),
    scratch_types=[
        pltpu.VMEM((_NT, _BPW), jnp.int32),
        pltpu.VMEM((_CV * _DP,), jnp.float32),
        pltpu.VMEM((_OW, _BPW), jnp.float32),
        pltpu.SemaphoreType.DMA,
    ],
    compiler_params=pltpu.CompilerParams(
        use_tc_tiling_on_sc=True,
        needs_layout_passes=False,
    ),
)


def kernel(x, y, emb0, emb1, emb2, emb3, emb4):
    tcat = jnp.concatenate([emb0, emb1, emb2, emb3, emb4], axis=0)
    tcat = jnp.pad(tcat, ((0, 0), (0, _DP - _D))).reshape(-1)
    out_t = _sc_call(x.T, y.astype(jnp.int32).T, tcat)
    return out_t.T
